# baseline (speedup 1.0000x reference)
import functools

import jax
import jax.numpy as jnp
from jax import lax
from jax.experimental import pallas as pl
from jax.experimental.pallas import tpu as pltpu
from jax.experimental.pallas import tpu_sc as plsc

N_NODES = 10000
N_EDGES = 320000
D = 128

NC = 2
NS = 16
NW = NC * NS

CHUNK = 128
TOT_CHUNKS = 2560
E_PAD = TOT_CHUNKS * CHUNK
S0 = 160
S1 = (TOT_CHUNKS // NS) - S0
N_PAD = 10240
ROWS_PER_TILE = N_PAD // NS


def _sc_segment_sum(x, eidx):
    mesh = plsc.VectorSubcoreMesh(core_axis_name="c", subcore_axis_name="s")

    @functools.partial(
        pl.kernel,
        out_type=jax.ShapeDtypeStruct((NC, N_PAD, D), jnp.float32),
        mesh=mesh,
        scratch_types=[
            pltpu.VMEM((4, 2, CHUNK), jnp.int32),
            pltpu.VMEM((2, CHUNK, D), jnp.float32),
            pltpu.VMEM_SHARED((N_PAD, D), jnp.float32),
        ] + [pltpu.SemaphoreType.DMA] * 6,
    )
    def run(x_hbm, e_hbm, out_hbm, idxb, rows, hacc, *sems):
        isem = sems[:4]
        gsem = sems[4:]
        c = lax.axis_index("c")
        s = lax.axis_index("s")

        zeros16 = jnp.zeros((16,), jnp.float32)

        def zero_row(i, _):
            for j in range(D // 16):
                rows[0, i, pl.ds(j * 16, 16)] = zeros16
            return 0

        lax.fori_loop(0, CHUNK, zero_row, 0)

        def zero_acc(k, _):
            pltpu.sync_copy(rows.at[0], hacc.at[pl.ds(s * ROWS_PER_TILE + k * CHUNK, CHUNK)])
            return 0

        lax.fori_loop(0, ROWS_PER_TILE // CHUNK, zero_acc, 0)
        plsc.subcore_barrier()

        def pipeline(steps, base):
            def stage_a(g, ki):
                pltpu.async_copy(e_hbm.at[base + g], idxb.at[ki], isem[ki])

            def stage_b(g, ki, b):
                pltpu.make_async_copy(e_hbm.at[base + g], idxb.at[ki], isem[ki]).wait()
                pltpu.async_copy(x_hbm.at[idxb.at[ki, 0]], rows.at[b], gsem[b])

            def stage_c(ki, b):
                pltpu.make_async_copy(x_hbm.at[idxb.at[ki, 0]], rows.at[b], gsem[b]).wait()
                pltpu.sync_copy(rows.at[b], hacc.at[idxb.at[ki, 1]], add=True)

            stage_a(0, 0)
            stage_a(1, 1)
            stage_b(0, 0, 0)
            stage_a(2, 2)
            stage_b(1, 1, 1)
            stage_a(3, 3)

            def it(i, _):
                g0 = i * 4
                for k in range(4):
                    g = g0 + k
                    stage_c(k, k % 2)
                    stage_a(g + 4, k)
                    stage_b(g + 2, (k + 2) % 4, k % 2)
                return 0

            lax.fori_loop(0, steps // 4 - 1, it, 0)

            stage_c(0, 0)
            stage_c(1, 1)
            stage_b(steps - 2, 2, 0)
            stage_b(steps - 1, 3, 1)
            stage_c(2, 0)
            stage_c(3, 1)

        @pl.when(c == 0)
        def _():
            pipeline(S0, s * S0)

        plsc.subcore_barrier()

        r0 = s * ROWS_PER_TILE
        pltpu.sync_copy(hacc.at[pl.ds(r0, ROWS_PER_TILE)],
                        out_hbm.at[c, pl.ds(r0, ROWS_PER_TILE)])

    return run(x, eidx)


def _tc_linear_body(h0_ref, h1_ref, wt_ref, b_ref, o_ref):
    h = h0_ref[...] + h1_ref[...]
    o_ref[...] = jnp.dot(h, wt_ref[...], preferred_element_type=jnp.float32) + b_ref[...]


def _tc_linear(h0, h1, wt, b):
    bm = 512
    return pl.pallas_call(
        _tc_linear_body,
        grid=(N_PAD // bm,),
        in_specs=[
            pl.BlockSpec((bm, D), lambda i: (i, 0)),
            pl.BlockSpec((bm, D), lambda i: (i, 0)),
            pl.BlockSpec((D, D), lambda i: (0, 0)),
            pl.BlockSpec((1, D), lambda i: (0, 0)),
        ],
        out_specs=pl.BlockSpec((bm, D), lambda i: (i, 0)),
        out_shape=jax.ShapeDtypeStruct((N_PAD, D), jnp.float32),
    )(h0, h1, wt, b)


def kernel(x, edge_index, W, b):
    ei = edge_index.astype(jnp.int32)
    pad = E_PAD - N_EDGES
    src = jnp.concatenate([ei[0], jnp.zeros((pad,), jnp.int32)])
    dst = jnp.concatenate([ei[1], jnp.full((pad,), N_NODES, jnp.int32)])
    eidx = jnp.stack([src.reshape(TOT_CHUNKS, CHUNK),
                      dst.reshape(TOT_CHUNKS, CHUNK)], axis=1)

    partials = _sc_segment_sum(x, eidx)
    out = _tc_linear(partials[0], partials[1], W.T, b.reshape(1, D))
    return out[:N_NODES]


# speedup vs baseline: 4.2097x; 4.2097x over previous
import functools

import jax
import jax.numpy as jnp
from jax import lax
from jax.experimental import pallas as pl
from jax.experimental.pallas import tpu as pltpu
from jax.experimental.pallas import tpu_sc as plsc

N_NODES = 10000
N_EDGES = 320000
D = 128

NC = 2
NS = 16
NW = NC * NS

CHUNK = 128
TOT_CHUNKS = 2560
E_PAD = TOT_CHUNKS * CHUNK
S0 = 80
S1 = (TOT_CHUNKS // NS) - S0
N_PAD = 10240
ROWS_PER_TILE = N_PAD // NS


def _sc_segment_sum(x, eidx):
    mesh = plsc.VectorSubcoreMesh(core_axis_name="c", subcore_axis_name="s")

    @functools.partial(
        pl.kernel,
        out_type=jax.ShapeDtypeStruct((NC, N_PAD, D), jnp.float32),
        mesh=mesh,
        scratch_types=[
            pltpu.VMEM((4, 2, CHUNK), jnp.int32),
            pltpu.VMEM((2, CHUNK, D), jnp.float32),
            pltpu.VMEM_SHARED((N_PAD, D), jnp.float32),
        ] + [pltpu.SemaphoreType.DMA] * 6,
    )
    def run(x_hbm, e_hbm, out_hbm, idxb, rows, hacc, *sems):
        isem = sems[:4]
        gsem = sems[4:]
        c = lax.axis_index("c")
        s = lax.axis_index("s")

        zeros16 = jnp.zeros((16,), jnp.float32)

        def zero_row(i, _):
            for j in range(D // 16):
                rows[0, i, pl.ds(j * 16, 16)] = zeros16
            return 0

        lax.fori_loop(0, CHUNK, zero_row, 0)

        def zero_acc(k, _):
            pltpu.sync_copy(rows.at[0], hacc.at[pl.ds(s * ROWS_PER_TILE + k * CHUNK, CHUNK)])
            return 0

        lax.fori_loop(0, ROWS_PER_TILE // CHUNK, zero_acc, 0)
        plsc.subcore_barrier()

        def pipeline(steps, base):
            def stage_a(g, ki):
                pltpu.async_copy(e_hbm.at[base + g], idxb.at[ki], isem[ki])

            def stage_b(g, ki, b):
                pltpu.make_async_copy(e_hbm.at[base + g], idxb.at[ki], isem[ki]).wait()
                pltpu.async_copy(x_hbm.at[idxb.at[ki, 0]], rows.at[b], gsem[b])

            def stage_c(ki, b):
                pltpu.make_async_copy(x_hbm.at[idxb.at[ki, 0]], rows.at[b], gsem[b]).wait()
                pltpu.sync_copy(rows.at[b], hacc.at[idxb.at[ki, 1]], add=True)

            stage_a(0, 0)
            stage_a(1, 1)
            stage_b(0, 0, 0)
            stage_a(2, 2)
            stage_b(1, 1, 1)
            stage_a(3, 3)

            def it(i, _):
                g0 = i * 4
                for k in range(4):
                    g = g0 + k
                    stage_c(k, k % 2)
                    stage_a(g + 4, k)
                    stage_b(g + 2, (k + 2) % 4, k % 2)
                return 0

            lax.fori_loop(0, steps // 4 - 1, it, 0)

            stage_c(0, 0)
            stage_c(1, 1)
            stage_b(steps - 2, 2, 0)
            stage_b(steps - 1, 3, 1)
            stage_c(2, 0)
            stage_c(3, 1)

        @pl.when(c == 0)
        def _():
            pipeline(S0, s * S0)

        @pl.when(c == 1)
        def _():
            pipeline(S1, NS * S0 + s * S1)

        plsc.subcore_barrier()

        r0 = s * ROWS_PER_TILE
        pltpu.sync_copy(hacc.at[pl.ds(r0, ROWS_PER_TILE)],
                        out_hbm.at[c, pl.ds(r0, ROWS_PER_TILE)])

    return run(x, eidx)


def _tc_linear_body(h0_ref, h1_ref, wt_ref, b_ref, o_ref):
    h = h0_ref[...] + h1_ref[...]
    o_ref[...] = jnp.dot(h, wt_ref[...], preferred_element_type=jnp.float32) + b_ref[...]


def _tc_linear(h0, h1, wt, b):
    bm = 512
    return pl.pallas_call(
        _tc_linear_body,
        grid=(N_PAD // bm,),
        in_specs=[
            pl.BlockSpec((bm, D), lambda i: (i, 0)),
            pl.BlockSpec((bm, D), lambda i: (i, 0)),
            pl.BlockSpec((D, D), lambda i: (0, 0)),
            pl.BlockSpec((1, D), lambda i: (0, 0)),
        ],
        out_specs=pl.BlockSpec((bm, D), lambda i: (i, 0)),
        out_shape=jax.ShapeDtypeStruct((N_PAD, D), jnp.float32),
    )(h0, h1, wt, b)


def kernel(x, edge_index, W, b):
    ei = edge_index.astype(jnp.int32)
    pad = E_PAD - N_EDGES
    fill = jnp.arange(pad, dtype=jnp.int32)
    src = jnp.concatenate([ei[0], fill % N_NODES])
    dst = jnp.concatenate([ei[1], N_NODES + fill % (N_PAD - N_NODES)])
    eidx = jnp.stack([src.reshape(TOT_CHUNKS, CHUNK),
                      dst.reshape(TOT_CHUNKS, CHUNK)], axis=1)

    partials = _sc_segment_sum(x, eidx)
    out = _tc_linear(partials[0], partials[1], W.T, b.reshape(1, D))
    return out[:N_NODES]


# speedup vs baseline: 4.5276x; 1.0755x over previous
import functools

import jax
import jax.numpy as jnp
from jax import lax
from jax.experimental import pallas as pl
from jax.experimental.pallas import tpu as pltpu
from jax.experimental.pallas import tpu_sc as plsc

N_NODES = 10000
N_EDGES = 320000
D = 128

NC = 2
NS = 16
NW = NC * NS

CHUNK = 128
TOT_CHUNKS = 2560
E_PAD = TOT_CHUNKS * CHUNK
S0 = 80
S1 = (TOT_CHUNKS // NS) - S0
N_PAD = 10240
ROWS_PER_TILE = N_PAD // NS


def _sc_segment_sum(x, srcp, dstp):
    mesh = plsc.VectorSubcoreMesh(core_axis_name="c", subcore_axis_name="s")

    @functools.partial(
        pl.kernel,
        out_type=jax.ShapeDtypeStruct((NC, N_PAD, D), jnp.float32),
        mesh=mesh,
        scratch_types=[
            pltpu.VMEM((4, 2, CHUNK), jnp.int32),
            pltpu.VMEM((2, CHUNK, D), jnp.float32),
            pltpu.VMEM_SHARED((N_PAD, D), jnp.float32),
        ] + [pltpu.SemaphoreType.DMA] * 6,
    )
    def run(x_hbm, src_hbm, dst_hbm, out_hbm, idxb, rows, hacc, *sems):
        isem = sems[:4]
        gsem = sems[4:]
        c = lax.axis_index("c")
        s = lax.axis_index("s")

        zeros16 = jnp.zeros((16,), jnp.float32)

        def zero_row(i, _):
            for j in range(D // 16):
                rows[0, i, pl.ds(j * 16, 16)] = zeros16
            return 0

        lax.fori_loop(0, CHUNK, zero_row, 0)

        def zero_acc(k, _):
            pltpu.sync_copy(rows.at[0], hacc.at[pl.ds(s * ROWS_PER_TILE + k * CHUNK, CHUNK)])
            return 0

        lax.fori_loop(0, ROWS_PER_TILE // CHUNK, zero_acc, 0)
        plsc.subcore_barrier()

        def pipeline(steps, base):
            def stage_a(g, ki):
                pltpu.async_copy(src_hbm.at[base + g], idxb.at[ki, 0], isem[ki])
                pltpu.async_copy(dst_hbm.at[base + g], idxb.at[ki, 1], isem[ki])

            def stage_b(g, ki, b):
                pltpu.make_async_copy(src_hbm.at[base + g], idxb.at[ki, 0], isem[ki]).wait()
                pltpu.make_async_copy(dst_hbm.at[base + g], idxb.at[ki, 1], isem[ki]).wait()
                pltpu.async_copy(x_hbm.at[idxb.at[ki, 0]], rows.at[b], gsem[b])

            def stage_c(ki, b):
                pltpu.make_async_copy(x_hbm.at[idxb.at[ki, 0]], rows.at[b], gsem[b]).wait()
                pltpu.sync_copy(rows.at[b], hacc.at[idxb.at[ki, 1]], add=True)

            stage_a(0, 0)
            stage_a(1, 1)
            stage_b(0, 0, 0)
            stage_a(2, 2)
            stage_b(1, 1, 1)
            stage_a(3, 3)

            def it(i, _):
                g0 = i * 4
                for k in range(4):
                    g = g0 + k
                    stage_c(k, k % 2)
                    stage_a(g + 4, k)
                    stage_b(g + 2, (k + 2) % 4, k % 2)
                return 0

            lax.fori_loop(0, steps // 4 - 1, it, 0)

            stage_c(0, 0)
            stage_c(1, 1)
            stage_b(steps - 2, 2, 0)
            stage_b(steps - 1, 3, 1)
            stage_c(2, 0)
            stage_c(3, 1)

        @pl.when(c == 0)
        def _():
            pipeline(S0, s * S0)

        @pl.when(c == 1)
        def _():
            pipeline(S1, NS * S0 + s * S1)

        plsc.subcore_barrier()

        r0 = s * ROWS_PER_TILE
        pltpu.sync_copy(hacc.at[pl.ds(r0, ROWS_PER_TILE)],
                        out_hbm.at[c, pl.ds(r0, ROWS_PER_TILE)])

    return run(x, srcp, dstp)


def _tc_linear_body(h0_ref, h1_ref, wt_ref, b_ref, o_ref):
    h = h0_ref[...] + h1_ref[...]
    o_ref[...] = jnp.dot(h, wt_ref[...], preferred_element_type=jnp.float32) + b_ref[...]


def _tc_linear(h0, h1, wt, b):
    bm = 1000
    return pl.pallas_call(
        _tc_linear_body,
        grid=(N_NODES // bm,),
        in_specs=[
            pl.BlockSpec((bm, D), lambda i: (i, 0)),
            pl.BlockSpec((bm, D), lambda i: (i, 0)),
            pl.BlockSpec((D, D), lambda i: (0, 0)),
            pl.BlockSpec((1, D), lambda i: (0, 0)),
        ],
        out_specs=pl.BlockSpec((bm, D), lambda i: (i, 0)),
        out_shape=jax.ShapeDtypeStruct((N_NODES, D), jnp.float32),
    )(h0, h1, wt, b)


def kernel(x, edge_index, W, b):
    ei = edge_index.astype(jnp.int32)
    pad = E_PAD - N_EDGES
    fill = jnp.arange(pad, dtype=jnp.int32)
    srcp = jnp.concatenate([ei[0], fill % N_NODES]).reshape(TOT_CHUNKS, CHUNK)
    dstp = jnp.concatenate([ei[1], N_NODES + fill % (N_PAD - N_NODES)]).reshape(TOT_CHUNKS, CHUNK)

    partials = _sc_segment_sum(x, srcp, dstp)
    return _tc_linear(partials[0], partials[1], W.T, b.reshape(1, D))


# speedup vs baseline: 5.0230x; 1.1094x over previous
import functools

import jax
import jax.numpy as jnp
from jax import lax
from jax.experimental import pallas as pl
from jax.experimental.pallas import tpu as pltpu
from jax.experimental.pallas import tpu_sc as plsc

N_NODES = 10000
N_EDGES = 320000
D = 128

NC = 2
NS = 16
NW = NC * NS

CHUNK = 128
TOT_CHUNKS = N_EDGES // CHUNK
STEPS = 80
N_PAD = 10240
ROWS_PER_TILE = N_PAD // NS


def _sc_segment_sum(x, ei):
    mesh = plsc.VectorSubcoreMesh(core_axis_name="c", subcore_axis_name="s")

    @functools.partial(
        pl.kernel,
        out_type=jax.ShapeDtypeStruct((NC, N_PAD, D), jnp.float32),
        mesh=mesh,
        scratch_types=[
            pltpu.VMEM((4, 2, CHUNK), jnp.int32),
            pltpu.VMEM((2, CHUNK, D), jnp.float32),
            pltpu.VMEM_SHARED((N_PAD, D), jnp.float32),
        ] + [pltpu.SemaphoreType.DMA] * 6,
    )
    def run(x_hbm, e_hbm, out_hbm, idxb, rows, hacc, *sems):
        isem = sems[:4]
        gsem = sems[4:]
        c = lax.axis_index("c")
        s = lax.axis_index("s")
        w = s * NC + c

        zeros16 = jnp.zeros((16,), jnp.float32)

        def zero_row(i, _):
            for j in range(D // 16):
                rows[0, i, pl.ds(j * 16, 16)] = zeros16
            return 0

        lax.fori_loop(0, CHUNK, zero_row, 0)

        def zero_acc(k, _):
            pltpu.sync_copy(rows.at[0], hacc.at[pl.ds(s * ROWS_PER_TILE + k * CHUNK, CHUNK)])
            return 0

        lax.fori_loop(0, ROWS_PER_TILE // CHUNK, zero_acc, 0)
        plsc.subcore_barrier()

        def edge_off(g):
            return (w + NW * g) * CHUNK

        def live(g):
            return w + NW * g < TOT_CHUNKS

        def stage_a(g, ki):
            pltpu.async_copy(e_hbm.at[0, pl.ds(edge_off(g), CHUNK)], idxb.at[ki, 0], isem[ki])
            pltpu.async_copy(e_hbm.at[1, pl.ds(edge_off(g), CHUNK)], idxb.at[ki, 1], isem[ki])

        def stage_b(g, ki, b):
            pltpu.make_async_copy(e_hbm.at[0, pl.ds(edge_off(g), CHUNK)], idxb.at[ki, 0], isem[ki]).wait()
            pltpu.make_async_copy(e_hbm.at[1, pl.ds(edge_off(g), CHUNK)], idxb.at[ki, 1], isem[ki]).wait()
            pltpu.async_copy(x_hbm.at[idxb.at[ki, 0]], rows.at[b], gsem[b])

        def stage_c(ki, b):
            pltpu.make_async_copy(x_hbm.at[idxb.at[ki, 0]], rows.at[b], gsem[b]).wait()
            pltpu.sync_copy(rows.at[b], hacc.at[idxb.at[ki, 1]], add=True)

        stage_a(0, 0)
        stage_a(1, 1)
        stage_b(0, 0, 0)
        stage_a(2, 2)
        stage_b(1, 1, 1)
        stage_a(3, 3)

        def it(i, _):
            g0 = i * 4
            for k in range(4):
                g = g0 + k
                stage_c(k, k % 2)

                @pl.when(live(g + 4))
                def _():
                    stage_a(g + 4, k)

                stage_b(g + 2, (k + 2) % 4, k % 2)
            return 0

        lax.fori_loop(0, STEPS // 4 - 1, it, 0)

        stage_c(0, 0)
        stage_c(1, 1)

        @pl.when(live(STEPS - 2))
        def _():
            stage_b(STEPS - 2, 2, 0)

        @pl.when(live(STEPS - 1))
        def _():
            stage_b(STEPS - 1, 3, 1)

        @pl.when(live(STEPS - 2))
        def _():
            stage_c(2, 0)

        @pl.when(live(STEPS - 1))
        def _():
            stage_c(3, 1)

        plsc.subcore_barrier()

        r0 = s * ROWS_PER_TILE
        pltpu.sync_copy(hacc.at[pl.ds(r0, ROWS_PER_TILE)],
                        out_hbm.at[c, pl.ds(r0, ROWS_PER_TILE)])

    return run(x, ei)


def _tc_linear_body(h0_ref, h1_ref, wt_ref, b_ref, o_ref):
    h = h0_ref[...] + h1_ref[...]
    o_ref[...] = jnp.dot(h, wt_ref[...], preferred_element_type=jnp.float32) + b_ref[...]


def _tc_linear(h0, h1, wt, b):
    bm = 1000
    return pl.pallas_call(
        _tc_linear_body,
        grid=(N_NODES // bm,),
        in_specs=[
            pl.BlockSpec((bm, D), lambda i: (i, 0)),
            pl.BlockSpec((bm, D), lambda i: (i, 0)),
            pl.BlockSpec((D, D), lambda i: (0, 0)),
            pl.BlockSpec((1, D), lambda i: (0, 0)),
        ],
        out_specs=pl.BlockSpec((bm, D), lambda i: (i, 0)),
        out_shape=jax.ShapeDtypeStruct((N_NODES, D), jnp.float32),
    )(h0, h1, wt, b)


def kernel(x, edge_index, W, b):
    partials = _sc_segment_sum(x, edge_index.astype(jnp.int32))
    return _tc_linear(partials[0], partials[1], W.T, b.reshape(1, D))
